# Initial kernel scaffold; baseline (speedup 1.0000x reference)
import jax
import jax.numpy as jnp
from jax.experimental import pallas as pl


def kernel(x, edge_index, W1l, b1l, W1r, W2l, b2l, W2r, Wo, bo):
    raise NotImplementedError("write your pallas kernel here")


# speedup vs baseline: 5.8621x; 5.8621x over previous
import jax
import jax.numpy as jnp
from jax import lax
from jax.experimental import pallas as pl
from jax.experimental.pallas import tpu as pltpu
from jax.experimental.pallas import tpu_sc as plsc

N = 10000
E = 320000
D = 128

NC = 2
NS = 16
NH = N // NC
DUMMY = NH
ACCR = 5008
EPT = E // NS
C = 80
NCHUNK = EPT // C
RPT = 312
RPT_LAST = NH - (NS - 1) * RPT
HR = 80
HPT = HR // NS


def _seg_sum_body(with_cnt, *refs):
    if with_cnt:
        (y, src_hbm, dst_hbm, zrow_hbm, p_hbm, cnt_hbm,
         src_all, dst_all, ldst_a, ldst_b, rows_a, rows_b, hist_v,
         acc, sem_ga, sem_gb, sem_sa, sem_sb) = refs
    else:
        (y, src_hbm, dst_hbm, zrow_hbm, p_hbm,
         src_all, dst_all, ldst_a, ldst_b, rows_a, rows_b,
         acc, sem_ga, sem_gb, sem_sa, sem_sb) = refs

    c = lax.axis_index("c")
    s = lax.axis_index("s")
    ebase = s * EPT
    rbase = s * RPT
    nodebase = c * NH

    @pl.when(s < NS - 1)
    def _():
        pltpu.sync_copy(zrow_hbm.at[pl.ds(0, RPT)], acc.at[pl.ds(rbase, RPT)])

    @pl.when(s == NS - 1)
    def _():
        pltpu.sync_copy(zrow_hbm, acc.at[pl.ds(rbase, RPT_LAST)])

    if with_cnt:
        zeros16 = jnp.zeros((16,), jnp.float32)

        def zfill(i, carry):
            for k in range(D // 16):
                hist_v[i, pl.ds(k * 16, 16)] = zeros16
            return carry

        lax.fori_loop(0, HR, zfill, 0)
    plsc.subcore_barrier()

    pltpu.sync_copy(src_hbm.at[pl.ds(ebase, EPT)], src_all)
    pltpu.sync_copy(dst_hbm.at[pl.ds(ebase, EPT)], dst_all)
    pltpu.async_copy(y.at[src_all.at[pl.ds(0, C)]], rows_a, sem_ga)

    def half_iter(g, rows_p, ldst_p, sem_gp, sem_sp, rows_q, ldst_q,
                  sem_gq, sem_sq):
        for k in range(C // 16):
            dv = dst_all[pl.ds(g * C + k * 16, 16)]
            lofs = dv - nodebase
            ok = jnp.logical_and(lofs >= 0, lofs < NH)
            ldst_p[pl.ds(k * 16, 16)] = jnp.where(ok, lofs, DUMMY)
            if with_cnt:
                cnts, last = plsc.scan_count(dv)
                plsc.addupdate_scatter(
                    hist_v, [lax.shift_right_logical(dv, 7),
                             lax.bitwise_and(dv, 127)],
                    cnts.astype(jnp.float32), mask=last)
        pltpu.make_async_copy(y.at[src_all.at[pl.ds(0, C)]],
                              rows_p, sem_gp).wait()
        pltpu.async_copy(rows_p, acc.at[ldst_p], sem_sp, add=True)

        @pl.when(g >= 1)
        def _():
            pltpu.make_async_copy(rows_q, acc.at[ldst_q], sem_sq).wait()

        @pl.when(g + 1 < NCHUNK)
        def _():
            pltpu.async_copy(y.at[src_all.at[pl.ds((g + 1) * C, C)]],
                             rows_q, sem_gq)

    def body(g, carry):
        even = lax.rem(g, 2) == 0

        @pl.when(even)
        def _():
            half_iter(g, rows_a, ldst_a, sem_ga, sem_sa,
                      rows_b, ldst_b, sem_gb, sem_sb)

        @pl.when(jnp.logical_not(even))
        def _():
            half_iter(g, rows_b, ldst_b, sem_gb, sem_sb,
                      rows_a, ldst_a, sem_ga, sem_sa)
        return carry

    lax.fori_loop(0, NCHUNK, body, 0)
    pltpu.make_async_copy(rows_b, acc.at[ldst_b], sem_sb).wait()
    if with_cnt:
        pltpu.sync_copy(hist_v, cnt_hbm.at[c, s])
    plsc.subcore_barrier()

    @pl.when(s < NS - 1)
    def _():
        pltpu.sync_copy(acc.at[pl.ds(rbase, RPT)],
                        p_hbm.at[c, pl.ds(rbase, RPT)])

    @pl.when(s == NS - 1)
    def _():
        pltpu.sync_copy(acc.at[pl.ds(rbase, RPT_LAST)],
                        p_hbm.at[c, pl.ds(rbase, RPT_LAST)])


def _make_seg_sum(with_cnt):
    import functools
    mesh = plsc.VectorSubcoreMesh(core_axis_name="c", subcore_axis_name="s")
    out_type = [jax.ShapeDtypeStruct((NC, NH, D), jnp.float32)]
    scratch = [
        pltpu.VMEM((EPT,), jnp.int32),
        pltpu.VMEM((EPT,), jnp.int32),
        pltpu.VMEM((C,), jnp.int32),
        pltpu.VMEM((C,), jnp.int32),
        pltpu.VMEM((C, D), jnp.float32),
        pltpu.VMEM((C, D), jnp.float32),
    ]
    if with_cnt:
        out_type.append(jax.ShapeDtypeStruct((NC, NS, HR, D), jnp.float32))
        scratch.append(pltpu.VMEM((HR, D), jnp.float32))
    scratch.append(pltpu.VMEM_SHARED((ACCR, D), jnp.float32))
    scratch += [pltpu.SemaphoreType.DMA, pltpu.SemaphoreType.DMA,
                pltpu.SemaphoreType.DMA, pltpu.SemaphoreType.DMA]

    return pl.kernel(
        functools.partial(_seg_sum_body, with_cnt),
        out_type=out_type, mesh=mesh,
        scratch_types=scratch,
        compiler_params=pltpu.CompilerParams(needs_layout_passes=False))


_seg_sum_cnt = _make_seg_sum(True)
_seg_sum = _make_seg_sum(False)


def _cntred_body(h_ref, o_ref):
    a = h_ref[0]
    for t in range(1, NS):
        a = a + h_ref[t]
    o_ref[...] = a


def _cntred(h):
    return pl.pallas_call(
        _cntred_body,
        out_shape=jax.ShapeDtypeStruct((HR, D), jnp.float32),
    )(h)


def _mid_body(p_ref, cnt_ref, x_ref, wl_ref, wr_ref, b_ref, o_ref):
    cnt = jnp.maximum(cnt_ref[...], 1.0)
    m = p_ref[...] / cnt
    z = (jnp.dot(m, wl_ref[...], preferred_element_type=jnp.float32)
         + jnp.dot(x_ref[...], wr_ref[...], preferred_element_type=jnp.float32)
         + b_ref[...])
    o_ref[...] = jnp.maximum(z, 0.0)


def _out_body(p_ref, cnt_ref, h_ref, wl_ref, wr_ref, b_ref, wo_ref, bo_ref,
              o_ref):
    cnt = jnp.maximum(cnt_ref[...], 1.0)
    m = p_ref[...] / cnt
    z = (jnp.dot(m, wl_ref[...], preferred_element_type=jnp.float32)
         + jnp.dot(h_ref[...], wr_ref[...], preferred_element_type=jnp.float32)
         + b_ref[...])
    h2 = jnp.maximum(z, 0.0)
    z2 = jnp.dot(h2, wo_ref[...], preferred_element_type=jnp.float32)
    o_ref[...] = jax.nn.sigmoid(z2 + bo_ref[...])


_GRID = 10
_BR = N // _GRID


def _mid(p, cnt, x, wl, wr, b):
    return pl.pallas_call(
        _mid_body,
        grid=(_GRID,),
        in_specs=[pl.BlockSpec((_BR, D), lambda i: (i, 0)),
                  pl.BlockSpec((_BR, 1), lambda i: (i, 0)),
                  pl.BlockSpec((_BR, D), lambda i: (i, 0)),
                  pl.BlockSpec((D, D), lambda i: (0, 0)),
                  pl.BlockSpec((D, D), lambda i: (0, 0)),
                  pl.BlockSpec((1, D), lambda i: (0, 0))],
        out_specs=pl.BlockSpec((_BR, D), lambda i: (i, 0)),
        out_shape=jax.ShapeDtypeStruct((N, D), jnp.float32),
    )(p, cnt, x, wl, wr, b)


def _final(p, cnt, h, wl, wr, b, wo, bo):
    return pl.pallas_call(
        _out_body,
        grid=(_GRID,),
        in_specs=[pl.BlockSpec((_BR, D), lambda i: (i, 0)),
                  pl.BlockSpec((_BR, 1), lambda i: (i, 0)),
                  pl.BlockSpec((_BR, D), lambda i: (i, 0)),
                  pl.BlockSpec((D, D), lambda i: (0, 0)),
                  pl.BlockSpec((D, D), lambda i: (0, 0)),
                  pl.BlockSpec((1, D), lambda i: (0, 0)),
                  pl.BlockSpec((D, D), lambda i: (0, 0)),
                  pl.BlockSpec((1, D), lambda i: (0, 0))],
        out_specs=pl.BlockSpec((_BR, D), lambda i: (i, 0)),
        out_shape=jax.ShapeDtypeStruct((N, D), jnp.float32),
    )(p, cnt, h, wl, wr, b, wo, bo)


@jax.jit
def kernel(x, edge_index, W1l, b1l, W1r, W2l, b2l, W2r, Wo, bo):
    src = edge_index[0]
    dst = edge_index[1]
    zrow = jnp.zeros((RPT_LAST, D), jnp.float32)

    p1c, hists = _seg_sum_cnt(x, src, dst, zrow)
    p1 = p1c.reshape(N, D)
    cnt = _cntred(hists[0]).reshape(HR * D)[:N].reshape(N, 1)
    h1 = _mid(p1, cnt, x, W1l.T, W1r.T, b1l.reshape(1, D))
    (p2c,) = _seg_sum(h1, src, dst, zrow)
    p2 = p2c.reshape(N, D)
    return _final(p2, cnt, h1, W2l.T, W2r.T, b2l.reshape(1, D), Wo.T,
                  bo.reshape(1, D))


# speedup vs baseline: 5.8706x; 1.0014x over previous
import jax
import jax.numpy as jnp
from jax import lax
from jax.experimental import pallas as pl
from jax.experimental.pallas import tpu as pltpu
from jax.experimental.pallas import tpu_sc as plsc

N = 10000
E = 320000
D = 128

NC = 2
NS = 16
NH = N // NC
DUMMY = NH
ACCR = 5008
EPT = E // NS
C = 80
NCHUNK = EPT // C
RPT = 312
RPT_LAST = NH - (NS - 1) * RPT
HR = 80


def _seg_sum_body(with_cnt, *refs):
    if with_cnt:
        (y, src_hbm, dst_hbm, zrow_hbm, p_hbm, cnt_hbm,
         src_all, dst_all, ldst_a, ldst_b, rows_a, rows_b, hist_v,
         acc, sem_ga, sem_gb, sem_sa, sem_sb) = refs
    else:
        (y, src_hbm, dst_hbm, zrow_hbm, p_hbm,
         src_all, dst_all, ldst_a, ldst_b, rows_a, rows_b,
         acc, sem_ga, sem_gb, sem_sa, sem_sb) = refs

    c = lax.axis_index("c")
    s = lax.axis_index("s")
    ebase = s * EPT
    rbase = s * RPT
    nodebase = c * NH

    @pl.when(s < NS - 1)
    def _():
        pltpu.sync_copy(zrow_hbm.at[pl.ds(0, RPT)], acc.at[pl.ds(rbase, RPT)])

    @pl.when(s == NS - 1)
    def _():
        pltpu.sync_copy(zrow_hbm, acc.at[pl.ds(rbase, RPT_LAST)])

    if with_cnt:
        zeros16 = jnp.zeros((16,), jnp.float32)

        def zfill(i, carry):
            for k in range(D // 16):
                hist_v[i, pl.ds(k * 16, 16)] = zeros16
            return carry

        lax.fori_loop(0, HR, zfill, 0)
    plsc.subcore_barrier()

    pltpu.sync_copy(src_hbm.at[pl.ds(ebase, EPT)], src_all)
    pltpu.sync_copy(dst_hbm.at[pl.ds(ebase, EPT)], dst_all)
    pltpu.async_copy(y.at[src_all.at[pl.ds(0, C)]], rows_a, sem_ga)

    def half_iter(g, rows_p, ldst_p, sem_gp, sem_sp, rows_q, ldst_q,
                  sem_gq, sem_sq):
        for k in range(C // 16):
            dv = dst_all[pl.ds(g * C + k * 16, 16)]
            lofs = dv - nodebase
            ok = jnp.logical_and(lofs >= 0, lofs < NH)
            ldst_p[pl.ds(k * 16, 16)] = jnp.where(ok, lofs, DUMMY)
            if with_cnt:
                cnts, last = plsc.scan_count(dv)
                plsc.addupdate_scatter(
                    hist_v, [lax.shift_right_logical(dv, 7),
                             lax.bitwise_and(dv, 127)],
                    cnts.astype(jnp.float32), mask=last)
        pltpu.make_async_copy(y.at[src_all.at[pl.ds(0, C)]],
                              rows_p, sem_gp).wait()
        pltpu.async_copy(rows_p, acc.at[ldst_p], sem_sp, add=True)

        @pl.when(g >= 1)
        def _():
            pltpu.make_async_copy(rows_q, acc.at[ldst_q], sem_sq).wait()

        @pl.when(g + 1 < NCHUNK)
        def _():
            pltpu.async_copy(y.at[src_all.at[pl.ds((g + 1) * C, C)]],
                             rows_q, sem_gq)

    def body(g, carry):
        even = lax.rem(g, 2) == 0

        @pl.when(even)
        def _():
            half_iter(g, rows_a, ldst_a, sem_ga, sem_sa,
                      rows_b, ldst_b, sem_gb, sem_sb)

        @pl.when(jnp.logical_not(even))
        def _():
            half_iter(g, rows_b, ldst_b, sem_gb, sem_sb,
                      rows_a, ldst_a, sem_ga, sem_sa)
        return carry

    lax.fori_loop(0, NCHUNK, body, 0)
    pltpu.make_async_copy(rows_b, acc.at[ldst_b], sem_sb).wait()
    if with_cnt:
        pltpu.sync_copy(hist_v, cnt_hbm.at[c, s])
    plsc.subcore_barrier()

    @pl.when(s < NS - 1)
    def _():
        pltpu.sync_copy(acc.at[pl.ds(rbase, RPT)],
                        p_hbm.at[c, pl.ds(rbase, RPT)])

    @pl.when(s == NS - 1)
    def _():
        pltpu.sync_copy(acc.at[pl.ds(rbase, RPT_LAST)],
                        p_hbm.at[c, pl.ds(rbase, RPT_LAST)])


def _make_seg_sum(with_cnt):
    import functools
    mesh = plsc.VectorSubcoreMesh(core_axis_name="c", subcore_axis_name="s")
    out_type = [jax.ShapeDtypeStruct((NC, NH, D), jnp.float32)]
    scratch = [
        pltpu.VMEM((EPT,), jnp.int32),
        pltpu.VMEM((EPT,), jnp.int32),
        pltpu.VMEM((C,), jnp.int32),
        pltpu.VMEM((C,), jnp.int32),
        pltpu.VMEM((C, D), jnp.float32),
        pltpu.VMEM((C, D), jnp.float32),
    ]
    if with_cnt:
        out_type.append(jax.ShapeDtypeStruct((NC, NS, HR, D), jnp.float32))
        scratch.append(pltpu.VMEM((HR, D), jnp.float32))
    scratch.append(pltpu.VMEM_SHARED((ACCR, D), jnp.float32))
    scratch += [pltpu.SemaphoreType.DMA, pltpu.SemaphoreType.DMA,
                pltpu.SemaphoreType.DMA, pltpu.SemaphoreType.DMA]

    return pl.kernel(
        functools.partial(_seg_sum_body, with_cnt),
        out_type=out_type, mesh=mesh,
        scratch_types=scratch,
        compiler_params=pltpu.CompilerParams(needs_layout_passes=False))


_seg_sum_cnt = _make_seg_sum(True)
_seg_sum = _make_seg_sum(False)


def _cntred_body(h_ref, o_ref):
    a = h_ref[0]
    for t in range(1, NS):
        a = a + h_ref[t]
    o_ref[...] = a


def _cntred(h):
    return pl.pallas_call(
        _cntred_body,
        out_shape=jax.ShapeDtypeStruct((HR, D), jnp.float32),
    )(h)


def _mid_body(p_ref, cnt_ref, x_ref, wl_ref, wr_ref, b_ref, o_ref):
    cnt = jnp.maximum(cnt_ref[...], 1.0)
    m = p_ref[...] / cnt
    z = (jnp.dot(m, wl_ref[...], preferred_element_type=jnp.float32)
         + jnp.dot(x_ref[...], wr_ref[...], preferred_element_type=jnp.float32)
         + b_ref[...])
    o_ref[...] = jnp.maximum(z, 0.0)


def _out_body(p_ref, cnt_ref, h_ref, wl_ref, wr_ref, b_ref, wo_ref, bo_ref,
              o_ref):
    cnt = jnp.maximum(cnt_ref[...], 1.0)
    m = p_ref[...] / cnt
    z = (jnp.dot(m, wl_ref[...], preferred_element_type=jnp.float32)
         + jnp.dot(h_ref[...], wr_ref[...], preferred_element_type=jnp.float32)
         + b_ref[...])
    h2 = jnp.maximum(z, 0.0)
    z2 = jnp.dot(h2, wo_ref[...], preferred_element_type=jnp.float32)
    o_ref[...] = jax.nn.sigmoid(z2 + bo_ref[...])


_GRID = 10
_BR = N // _GRID


def _mid(p, cnt, x, wl, wr, b):
    return pl.pallas_call(
        _mid_body,
        grid=(_GRID,),
        in_specs=[pl.BlockSpec((_BR, D), lambda i: (i, 0)),
                  pl.BlockSpec((_BR, 1), lambda i: (i, 0)),
                  pl.BlockSpec((_BR, D), lambda i: (i, 0)),
                  pl.BlockSpec((D, D), lambda i: (0, 0)),
                  pl.BlockSpec((D, D), lambda i: (0, 0)),
                  pl.BlockSpec((1, D), lambda i: (0, 0))],
        out_specs=pl.BlockSpec((_BR, D), lambda i: (i, 0)),
        out_shape=jax.ShapeDtypeStruct((N, D), jnp.float32),
    )(p, cnt, x, wl, wr, b)


def _final(p, cnt, h, wl, wr, b, wo, bo):
    return pl.pallas_call(
        _out_body,
        grid=(_GRID,),
        in_specs=[pl.BlockSpec((_BR, D), lambda i: (i, 0)),
                  pl.BlockSpec((_BR, 1), lambda i: (i, 0)),
                  pl.BlockSpec((_BR, D), lambda i: (i, 0)),
                  pl.BlockSpec((D, D), lambda i: (0, 0)),
                  pl.BlockSpec((D, D), lambda i: (0, 0)),
                  pl.BlockSpec((1, D), lambda i: (0, 0)),
                  pl.BlockSpec((D, D), lambda i: (0, 0)),
                  pl.BlockSpec((1, D), lambda i: (0, 0))],
        out_specs=pl.BlockSpec((_BR, D), lambda i: (i, 0)),
        out_shape=jax.ShapeDtypeStruct((N, D), jnp.float32),
    )(p, cnt, h, wl, wr, b, wo, bo)


@jax.jit
def kernel(x, edge_index, W1l, b1l, W1r, W2l, b2l, W2r, Wo, bo):
    src = edge_index[0]
    dst = edge_index[1]
    zrow = jnp.zeros((RPT_LAST, D), jnp.float32)

    p1c, hists = _seg_sum_cnt(x, src, dst, zrow)
    p1 = p1c.reshape(N, D)
    cnt = _cntred(hists[0]).reshape(HR * D)[:N].reshape(N, 1)
    h1 = _mid(p1, cnt, x, W1l.T, W1r.T, b1l.reshape(1, D))
    (p2c,) = _seg_sum(h1, src, dst, zrow)
    p2 = p2c.reshape(N, D)
    return _final(p2, cnt, h1, W2l.T, W2r.T, b2l.reshape(1, D), Wo.T,
                  bo.reshape(1, D))
